# baseline (speedup 1.0000x reference)
import functools

import jax
import jax.numpy as jnp
from jax import lax
from jax.experimental import pallas as pl
from jax.experimental.pallas import tpu as pltpu


def _prep_body(W1, Wg1, W2, Wg2, W3, b1, bg1, b2, bg2, b3,
               A1, B1, A2, B2, c):
    w3a = W3[0:128, :]
    w3b = W3[128:256, :]
    w3c = W3[256:384, :]
    w3d = W3[384:512, :]
    A1[...] = jnp.dot(W1[...], w3a, preferred_element_type=jnp.float32)
    B1[...] = jnp.dot(Wg1[...], w3b, preferred_element_type=jnp.float32)
    A2[...] = jnp.dot(W2[...], w3c, preferred_element_type=jnp.float32)
    B2[...] = jnp.dot(Wg2[...], w3d, preferred_element_type=jnp.float32)
    c[...] = (b3[...]
              + jnp.dot(b1[...], w3a, preferred_element_type=jnp.float32)
              + jnp.dot(bg1[...], w3b, preferred_element_type=jnp.float32)
              + jnp.dot(b2[...], w3c, preferred_element_type=jnp.float32)
              + jnp.dot(bg2[...], w3d, preferred_element_type=jnp.float32))


def _prep(W1, Wg1, W2, Wg2, W3, b1, bg1, b2, bg2, b3):
    f32 = jnp.float32
    return pl.pallas_call(
        _prep_body,
        out_shape=[
            jax.ShapeDtypeStruct((128, 256), f32),
            jax.ShapeDtypeStruct((128, 256), f32),
            jax.ShapeDtypeStruct((128, 256), f32),
            jax.ShapeDtypeStruct((128, 256), f32),
            jax.ShapeDtypeStruct((1, 256), f32),
        ],
    )(W1, Wg1, W2, Wg2, W3,
      b1.reshape(1, 128), bg1.reshape(1, 128),
      b2.reshape(1, 128), bg2.reshape(1, 128), b3.reshape(1, 256))


def _main_body(te, f2, a1, a2, deg, A1, B1, A2, B2, c, out):
    r = 1.0 / jnp.maximum(deg[...], 1.0)
    acc = jnp.dot(te[...], A1[...], preferred_element_type=jnp.float32)
    acc += jnp.dot(f2[...], A2[...], preferred_element_type=jnp.float32)
    acc += jnp.dot(a1[...] * r, B1[...], preferred_element_type=jnp.float32)
    acc += jnp.dot(a2[...] * r, B2[...], preferred_element_type=jnp.float32)
    out[...] = acc + c[...]


def _fused_matmul(te, f2, agg1, agg2, deg, A1, B1, A2, B2, c):
    n = te.shape[0]
    R = 2000
    grid = (n // R,)
    row_blk = pl.BlockSpec((R, 128), lambda i: (i, 0))
    w_blk = pl.BlockSpec((128, 256), lambda i: (0, 0))
    return pl.pallas_call(
        _main_body,
        grid=grid,
        in_specs=[row_blk, row_blk, row_blk, row_blk,
                  pl.BlockSpec((R, 1), lambda i: (i, 0)),
                  w_blk, w_blk, w_blk, w_blk,
                  pl.BlockSpec((1, 256), lambda i: (0, 0))],
        out_specs=pl.BlockSpec((R, 256), lambda i: (i, 0)),
        out_shape=jax.ShapeDtypeStruct((n, 256), jnp.float32),
    )(te, f2, agg1, agg2, deg, A1, B1, A2, B2, c)


def kernel(text_emb, feature_2, edge_index, W_g1, b_g1, W_g2, b_g2,
           W1, b1, W2, b2, W3, b3):
    n = text_emb.shape[0]
    src = edge_index[0]
    dst = edge_index[1]
    agg1 = jax.ops.segment_sum(jnp.take(text_emb, src, axis=0), dst, num_segments=n)
    agg2 = jax.ops.segment_sum(jnp.take(feature_2, src, axis=0), dst, num_segments=n)
    deg = jax.ops.segment_sum(jnp.ones((src.shape[0],), jnp.float32), dst, num_segments=n)
    A1, B1, A2, B2, c = _prep(W1, W_g1, W2, W_g2, W3, b1, b_g1, b2, b_g2, b3)
    return _fused_matmul(text_emb, feature_2, agg1, agg2,
                         deg.reshape(n, 1), A1, B1, A2, B2, c)


# speedup vs baseline: 4.2334x; 4.2334x over previous
import functools

import jax
import jax.numpy as jnp
from jax import lax
from jax.experimental import pallas as pl
from jax.experimental.pallas import tpu as pltpu
from jax.experimental.pallas import tpu_sc as plsc

_NPAD = 10240
_RPT = 640
_CH = 128
_IB = 8


def _sc_aggregate(xcat, src3, dst3, n_chunks):
    f32 = jnp.float32
    mesh = plsc.VectorSubcoreMesh(core_axis_name="c", subcore_axis_name="s")

    @functools.partial(
        pl.kernel,
        out_type=jax.ShapeDtypeStruct((2, _NPAD, 128), f32),
        mesh=mesh,
        scratch_types=[
            pltpu.VMEM_SHARED((_NPAD, 128), f32),
            pltpu.VMEM((_IB, _CH), jnp.int32),
            pltpu.VMEM((_IB, _CH), jnp.int32),
            pltpu.VMEM((_CH, 128), f32),
            pltpu.SemaphoreType.DMA,
        ],
    )
    def k(x_hbm, src_hbm, dst_hbm, agg_out, acc, src_v, dst_v, gbuf, sem):
        c = lax.axis_index("c")
        s = lax.axis_index("s")
        row_base = s * _RPT
        chunk_base = s * n_chunks

        def fill_zero(i, carry):
            for q in range(8):
                gbuf[i, pl.ds(q * 16, 16)] = jnp.zeros((16,), f32)
            return carry
        lax.fori_loop(0, _CH, fill_zero, 0)

        def zero_acc(j, carry):
            pltpu.sync_copy(gbuf, acc.at[pl.ds(row_base + j * _CH, _CH)])
            return carry
        lax.fori_loop(0, _RPT // _CH, zero_acc, 0)

        plsc.subcore_barrier()

        def block(b, carry):
            pltpu.sync_copy(src_hbm.at[c, pl.ds(chunk_base + b * _IB, _IB)], src_v)
            pltpu.sync_copy(dst_hbm.at[pl.ds(chunk_base + b * _IB, _IB)], dst_v)
            for j in range(_IB):
                pltpu.async_copy(x_hbm.at[src_v.at[j]], gbuf, sem).wait()
                pltpu.sync_copy(gbuf, acc.at[dst_v.at[j]], add=True)
            return carry
        lax.fori_loop(0, n_chunks // _IB, block, 0)

        plsc.subcore_barrier()

        rows = pl.ds(row_base, _RPT)
        pltpu.sync_copy(acc.at[rows], agg_out.at[c, rows])

    return k(xcat, src3, dst3)


def _sc_degree(dst6, half_chunks):
    f32 = jnp.float32
    mesh = plsc.VectorSubcoreMesh(core_axis_name="c", subcore_axis_name="s")

    @functools.partial(
        pl.kernel,
        out_type=jax.ShapeDtypeStruct((2, _NPAD, 128), f32),
        mesh=mesh,
        scratch_types=[
            pltpu.VMEM_SHARED((_NPAD, 128), f32),
            pltpu.VMEM((_IB, _CH), jnp.int32),
            pltpu.VMEM((_CH, 128), f32),
        ],
    )
    def k(dst_hbm, deg_out, acc, dst_v, ones_v):
        c = lax.axis_index("c")
        s = lax.axis_index("s")
        row_base = s * _RPT
        chunk_base = s * half_chunks

        def fill(val):
            def body(i, carry):
                for q in range(8):
                    ones_v[i, pl.ds(q * 16, 16)] = jnp.full((16,), val, f32)
                return carry
            lax.fori_loop(0, _CH, body, 0)

        fill(0.0)

        def zero_acc(j, carry):
            pltpu.sync_copy(ones_v, acc.at[pl.ds(row_base + j * _CH, _CH)])
            return carry
        lax.fori_loop(0, _RPT // _CH, zero_acc, 0)

        fill(1.0)
        plsc.subcore_barrier()

        def block(b, carry):
            pltpu.sync_copy(dst_hbm.at[c, pl.ds(chunk_base + b * _IB, _IB)], dst_v)
            for j in range(_IB):
                pltpu.sync_copy(ones_v, acc.at[dst_v.at[j]], add=True)
            return carry
        lax.fori_loop(0, half_chunks // _IB, block, 0)

        plsc.subcore_barrier()

        rows = pl.ds(row_base, _RPT)
        pltpu.sync_copy(acc.at[rows], deg_out.at[c, rows])

    return k(dst6)


def _prep_body(W1, Wg1, W2, Wg2, W3, b1, bg1, b2, bg2, b3,
               A1, B1, A2, B2, c):
    w3a = W3[0:128, :]
    w3b = W3[128:256, :]
    w3c = W3[256:384, :]
    w3d = W3[384:512, :]
    A1[...] = jnp.dot(W1[...], w3a, preferred_element_type=jnp.float32)
    B1[...] = jnp.dot(Wg1[...], w3b, preferred_element_type=jnp.float32)
    A2[...] = jnp.dot(W2[...], w3c, preferred_element_type=jnp.float32)
    B2[...] = jnp.dot(Wg2[...], w3d, preferred_element_type=jnp.float32)
    c[...] = (b3[...]
              + jnp.dot(b1[...], w3a, preferred_element_type=jnp.float32)
              + jnp.dot(bg1[...], w3b, preferred_element_type=jnp.float32)
              + jnp.dot(b2[...], w3c, preferred_element_type=jnp.float32)
              + jnp.dot(bg2[...], w3d, preferred_element_type=jnp.float32))


def _prep(W1, Wg1, W2, Wg2, W3, b1, bg1, b2, bg2, b3):
    f32 = jnp.float32
    return pl.pallas_call(
        _prep_body,
        out_shape=[
            jax.ShapeDtypeStruct((128, 256), f32),
            jax.ShapeDtypeStruct((128, 256), f32),
            jax.ShapeDtypeStruct((128, 256), f32),
            jax.ShapeDtypeStruct((128, 256), f32),
            jax.ShapeDtypeStruct((1, 256), f32),
        ],
    )(W1, Wg1, W2, Wg2, W3,
      b1.reshape(1, 128), bg1.reshape(1, 128),
      b2.reshape(1, 128), bg2.reshape(1, 128), b3.reshape(1, 256))


def _main_body(te, f2, agg, deg, A1, B1, A2, B2, c, out):
    r = 1.0 / jnp.maximum(deg[0] + deg[1], 1.0)
    acc = jnp.dot(te[...], A1[...], preferred_element_type=jnp.float32)
    acc += jnp.dot(f2[...], A2[...], preferred_element_type=jnp.float32)
    acc += jnp.dot(agg[0] * r, B1[...], preferred_element_type=jnp.float32)
    acc += jnp.dot(agg[1] * r, B2[...], preferred_element_type=jnp.float32)
    out[...] = acc + c[...]


def _fused_matmul(te, f2, agg, deg, A1, B1, A2, B2, c):
    n = te.shape[0]
    R = 2000
    grid = (n // R,)
    row_blk = pl.BlockSpec((R, 128), lambda i: (i, 0))
    pair_blk = pl.BlockSpec((2, R, 128), lambda i: (0, i, 0))
    w_blk = pl.BlockSpec((128, 256), lambda i: (0, 0))
    return pl.pallas_call(
        _main_body,
        grid=grid,
        in_specs=[row_blk, row_blk, pair_blk, pair_blk,
                  w_blk, w_blk, w_blk, w_blk,
                  pl.BlockSpec((1, 256), lambda i: (0, 0))],
        out_specs=pl.BlockSpec((R, 256), lambda i: (i, 0)),
        out_shape=jax.ShapeDtypeStruct((n, 256), jnp.float32),
    )(te, f2, agg, deg, A1, B1, A2, B2, c)


def kernel(text_emb, feature_2, edge_index, W_g1, b_g1, W_g2, b_g2,
           W1, b1, W2, b2, W3, b3):
    n = text_emb.shape[0]
    e = edge_index.shape[1]
    src = edge_index[0].astype(jnp.int32)
    dst = edge_index[1].astype(jnp.int32)

    n_chunks = -(-e // (16 * _CH))
    n_chunks = (n_chunks + 15) // 16 * 16
    epad = 16 * n_chunks * _CH
    src_p = jnp.concatenate([src, jnp.zeros((epad - e,), jnp.int32)])
    dst_p = jnp.concatenate([dst, jnp.full((epad - e,), n, jnp.int32)])
    src3 = jnp.stack([src_p, src_p + n]).reshape(2, 16 * n_chunks, _CH)
    dst3 = dst_p.reshape(16 * n_chunks, _CH)
    dst6 = dst_p.reshape(2, 8 * n_chunks, _CH)
    xcat = jnp.concatenate([text_emb, feature_2], axis=0)

    agg = _sc_aggregate(xcat, src3, dst3, n_chunks)
    deg = _sc_degree(dst6, n_chunks // 2)

    A1, B1, A2, B2, c = _prep(W1, W_g1, W2, W_g2, W3, b1, b_g1, b2, b_g2, b3)
    return _fused_matmul(text_emb, feature_2, agg, deg, A1, B1, A2, B2, c)


# speedup vs baseline: 5.2085x; 1.2303x over previous
import functools

import jax
import jax.numpy as jnp
from jax import lax
from jax.experimental import pallas as pl
from jax.experimental.pallas import tpu as pltpu
from jax.experimental.pallas import tpu_sc as plsc

_NPAD = 10240
_RPT = 640
_CH = 128
_IB = 8
_AB = 32


def _sc_aggregate(xcat, src3, dst3, n_chunks):
    f32 = jnp.float32
    mesh = plsc.VectorSubcoreMesh(core_axis_name="c", subcore_axis_name="s")

    @functools.partial(
        pl.kernel,
        out_type=jax.ShapeDtypeStruct((2, _NPAD, 128), f32),
        mesh=mesh,
        scratch_types=[
            pltpu.VMEM_SHARED((_NPAD, 128), f32),
            pltpu.VMEM((_AB, _CH), jnp.int32),
            pltpu.VMEM((_AB, _CH), jnp.int32),
            pltpu.VMEM((_CH, 128), f32),
            pltpu.VMEM((_CH, 128), f32),
            pltpu.SemaphoreType.DMA,
            pltpu.SemaphoreType.DMA,
        ],
    )
    def k(x_hbm, src_hbm, dst_hbm, agg_out,
          acc, src_v, dst_v, gbuf0, gbuf1, sem0, sem1):
        c = lax.axis_index("c")
        s = lax.axis_index("s")
        row_base = s * _RPT
        chunk_base = s * n_chunks

        def fill_zero(i, carry):
            for q in range(8):
                gbuf0[i, pl.ds(q * 16, 16)] = jnp.zeros((16,), f32)
            return carry
        lax.fori_loop(0, _CH, fill_zero, 0)

        def zero_acc(j, carry):
            pltpu.sync_copy(gbuf0, acc.at[pl.ds(row_base + j * _CH, _CH)])
            return carry
        lax.fori_loop(0, _RPT // _CH, zero_acc, 0)

        plsc.subcore_barrier()

        def block(b, carry):
            pltpu.sync_copy(src_hbm.at[c, pl.ds(chunk_base + b * _AB, _AB)], src_v)
            pltpu.sync_copy(dst_hbm.at[pl.ds(chunk_base + b * _AB, _AB)], dst_v)
            pltpu.async_copy(x_hbm.at[src_v.at[0]], gbuf0, sem0)

            def pair(p, carry2):
                j = p * 2
                pltpu.async_copy(x_hbm.at[src_v.at[j + 1]], gbuf1, sem1)
                pltpu.make_async_copy(x_hbm.at[src_v.at[j]], gbuf0, sem0).wait()
                pltpu.sync_copy(gbuf0, acc.at[dst_v.at[j]], add=True)

                @pl.when(p < _AB // 2 - 1)
                def _():
                    pltpu.async_copy(x_hbm.at[src_v.at[j + 2]], gbuf0, sem0)
                pltpu.make_async_copy(x_hbm.at[src_v.at[j + 1]], gbuf1, sem1).wait()
                pltpu.sync_copy(gbuf1, acc.at[dst_v.at[j + 1]], add=True)
                return carry2
            lax.fori_loop(0, _AB // 2, pair, 0)
            return carry
        lax.fori_loop(0, n_chunks // _AB, block, 0)

        plsc.subcore_barrier()

        rows = pl.ds(row_base, _RPT)
        pltpu.sync_copy(acc.at[rows], agg_out.at[c, rows])

    return k(xcat, src3, dst3)


def _sc_degree(dst6, half_chunks):
    f32 = jnp.float32
    mesh = plsc.VectorSubcoreMesh(core_axis_name="c", subcore_axis_name="s")

    @functools.partial(
        pl.kernel,
        out_type=jax.ShapeDtypeStruct((2, _NPAD, 128), f32),
        mesh=mesh,
        scratch_types=[
            pltpu.VMEM_SHARED((_NPAD, 128), f32),
            pltpu.VMEM((_IB, _CH), jnp.int32),
            pltpu.VMEM((_CH, 128), f32),
        ],
    )
    def k(dst_hbm, deg_out, acc, dst_v, ones_v):
        c = lax.axis_index("c")
        s = lax.axis_index("s")
        row_base = s * _RPT
        chunk_base = s * half_chunks

        def fill(val):
            def body(i, carry):
                for q in range(8):
                    ones_v[i, pl.ds(q * 16, 16)] = jnp.full((16,), val, f32)
                return carry
            lax.fori_loop(0, _CH, body, 0)

        fill(0.0)

        def zero_acc(j, carry):
            pltpu.sync_copy(ones_v, acc.at[pl.ds(row_base + j * _CH, _CH)])
            return carry
        lax.fori_loop(0, _RPT // _CH, zero_acc, 0)

        fill(1.0)
        plsc.subcore_barrier()

        def block(b, carry):
            pltpu.sync_copy(dst_hbm.at[c, pl.ds(chunk_base + b * _IB, _IB)], dst_v)
            for j in range(_IB):
                pltpu.sync_copy(ones_v, acc.at[dst_v.at[j]], add=True)
            return carry
        lax.fori_loop(0, half_chunks // _IB, block, 0)

        plsc.subcore_barrier()

        rows = pl.ds(row_base, _RPT)
        pltpu.sync_copy(acc.at[rows], deg_out.at[c, rows])

    return k(dst6)


def _prep_body(W1, Wg1, W2, Wg2, W3, b1, bg1, b2, bg2, b3,
               A1, B1, A2, B2, c):
    w3a = W3[0:128, :]
    w3b = W3[128:256, :]
    w3c = W3[256:384, :]
    w3d = W3[384:512, :]
    A1[...] = jnp.dot(W1[...], w3a, preferred_element_type=jnp.float32)
    B1[...] = jnp.dot(Wg1[...], w3b, preferred_element_type=jnp.float32)
    A2[...] = jnp.dot(W2[...], w3c, preferred_element_type=jnp.float32)
    B2[...] = jnp.dot(Wg2[...], w3d, preferred_element_type=jnp.float32)
    c[...] = (b3[...]
              + jnp.dot(b1[...], w3a, preferred_element_type=jnp.float32)
              + jnp.dot(bg1[...], w3b, preferred_element_type=jnp.float32)
              + jnp.dot(b2[...], w3c, preferred_element_type=jnp.float32)
              + jnp.dot(bg2[...], w3d, preferred_element_type=jnp.float32))


def _prep(W1, Wg1, W2, Wg2, W3, b1, bg1, b2, bg2, b3):
    f32 = jnp.float32
    return pl.pallas_call(
        _prep_body,
        out_shape=[
            jax.ShapeDtypeStruct((128, 256), f32),
            jax.ShapeDtypeStruct((128, 256), f32),
            jax.ShapeDtypeStruct((128, 256), f32),
            jax.ShapeDtypeStruct((128, 256), f32),
            jax.ShapeDtypeStruct((1, 256), f32),
        ],
    )(W1, Wg1, W2, Wg2, W3,
      b1.reshape(1, 128), bg1.reshape(1, 128),
      b2.reshape(1, 128), bg2.reshape(1, 128), b3.reshape(1, 256))


def _main_body(te, f2, agg, deg, A1, B1, A2, B2, c, out):
    r = 1.0 / jnp.maximum(deg[0] + deg[1], 1.0)
    acc = jnp.dot(te[...], A1[...], preferred_element_type=jnp.float32)
    acc += jnp.dot(f2[...], A2[...], preferred_element_type=jnp.float32)
    acc += jnp.dot(agg[0] * r, B1[...], preferred_element_type=jnp.float32)
    acc += jnp.dot(agg[1] * r, B2[...], preferred_element_type=jnp.float32)
    out[...] = acc + c[...]


def _fused_matmul(te, f2, agg, deg, A1, B1, A2, B2, c):
    n = te.shape[0]
    R = 2000
    grid = (n // R,)
    row_blk = pl.BlockSpec((R, 128), lambda i: (i, 0))
    pair_blk = pl.BlockSpec((2, R, 128), lambda i: (0, i, 0))
    w_blk = pl.BlockSpec((128, 256), lambda i: (0, 0))
    return pl.pallas_call(
        _main_body,
        grid=grid,
        in_specs=[row_blk, row_blk, pair_blk, pair_blk,
                  w_blk, w_blk, w_blk, w_blk,
                  pl.BlockSpec((1, 256), lambda i: (0, 0))],
        out_specs=pl.BlockSpec((R, 256), lambda i: (i, 0)),
        out_shape=jax.ShapeDtypeStruct((n, 256), jnp.float32),
    )(te, f2, agg, deg, A1, B1, A2, B2, c)


def kernel(text_emb, feature_2, edge_index, W_g1, b_g1, W_g2, b_g2,
           W1, b1, W2, b2, W3, b3):
    n = text_emb.shape[0]
    e = edge_index.shape[1]
    src = edge_index[0].astype(jnp.int32)
    dst = edge_index[1].astype(jnp.int32)

    n_chunks = -(-e // (16 * _CH))
    n_chunks = (n_chunks + _AB - 1) // _AB * _AB
    epad = 16 * n_chunks * _CH
    src_p = jnp.concatenate([src, jnp.zeros((epad - e,), jnp.int32)])
    dst_p = jnp.concatenate([dst, jnp.full((epad - e,), n, jnp.int32)])
    src3 = jnp.stack([src_p, src_p + n]).reshape(2, 16 * n_chunks, _CH)
    dst3 = dst_p.reshape(16 * n_chunks, _CH)
    dst6 = dst_p.reshape(2, 8 * n_chunks, _CH)
    xcat = jnp.concatenate([text_emb, feature_2], axis=0)

    agg = _sc_aggregate(xcat, src3, dst3, n_chunks)
    deg = _sc_degree(dst6, n_chunks // 2)

    A1, B1, A2, B2, c = _prep(W1, W_g1, W2, W_g2, W3, b1, b_g1, b2, b_g2, b3)
    return _fused_matmul(text_emb, feature_2, agg, deg, A1, B1, A2, B2, c)


# speedup vs baseline: 5.2976x; 1.0171x over previous
import functools

import jax
import jax.numpy as jnp
from jax import lax
from jax.experimental import pallas as pl
from jax.experimental.pallas import tpu as pltpu
from jax.experimental.pallas import tpu_sc as plsc

_NPAD = 10240
_RPT = 640
_CH = 128
_IB = 8
_AB = 32


def _sc_aggregate(xcat, src3, dst3, n_chunks):
    f32 = jnp.float32
    mesh = plsc.VectorSubcoreMesh(core_axis_name="c", subcore_axis_name="s")

    @functools.partial(
        pl.kernel,
        out_type=jax.ShapeDtypeStruct((2, _NPAD, 128), f32),
        mesh=mesh,
        scratch_types=[
            pltpu.VMEM_SHARED((_NPAD, 128), f32),
            pltpu.VMEM((_AB, _CH), jnp.int32),
            pltpu.VMEM((_AB, _CH), jnp.int32),
            pltpu.VMEM((64, 128), f32),
            pltpu.VMEM((64, 128), f32),
            pltpu.VMEM((64, 128), f32),
            pltpu.VMEM((64, 128), f32),
            pltpu.SemaphoreType.DMA,
            pltpu.SemaphoreType.DMA,
            pltpu.SemaphoreType.DMA,
            pltpu.SemaphoreType.DMA,
        ],
    )
    def k(x_hbm, src_hbm, dst_hbm, agg_out,
          acc, src_v, dst_v, gbuf0, gbuf1, gbuf2, gbuf3,
          sem0, sem1, sem2, sem3):
        c = lax.axis_index("c")
        s = lax.axis_index("s")
        row_base = s * _RPT
        chunk_base = s * n_chunks
        gbufs = [gbuf0, gbuf1, gbuf2, gbuf3]
        sems = [sem0, sem1, sem2, sem3]
        nh = 2 * _AB

        def fill_zero(i, carry):
            for q in range(8):
                gbuf0[i, pl.ds(q * 16, 16)] = jnp.zeros((16,), f32)
            return carry
        lax.fori_loop(0, 64, fill_zero, 0)

        def zero_acc(j, carry):
            pltpu.sync_copy(gbuf0, acc.at[pl.ds(row_base + j * 64, 64)])
            return carry
        lax.fori_loop(0, _RPT // 64, zero_acc, 0)

        plsc.subcore_barrier()

        def half_idx(k):
            return k // 2, (k % 2) * 64

        def issue(k, u):
            j, off = half_idx(k)
            pltpu.async_copy(
                x_hbm.at[src_v.at[j, pl.ds(off, 64)]], gbufs[u], sems[u])

        def block(b, carry):
            pltpu.sync_copy(src_hbm.at[c, pl.ds(chunk_base + b * _AB, _AB)], src_v)
            pltpu.sync_copy(dst_hbm.at[pl.ds(chunk_base + b * _AB, _AB)], dst_v)
            for u in range(4):
                issue(u, u)

            def quad(q, carry2):
                for u in range(4):
                    k = q * 4 + u
                    j, off = half_idx(k)
                    pltpu.make_async_copy(
                        x_hbm.at[src_v.at[j, pl.ds(off, 64)]],
                        gbufs[u], sems[u]).wait()
                    pltpu.sync_copy(
                        gbufs[u], acc.at[dst_v.at[j, pl.ds(off, 64)]],
                        add=True)

                    @pl.when(k + 4 < nh)
                    def _():
                        kk = k + 4
                        jj, off2 = half_idx(kk)
                        pltpu.async_copy(
                            x_hbm.at[src_v.at[jj, pl.ds(off2, 64)]],
                            gbufs[u], sems[u])
                return carry2
            lax.fori_loop(0, nh // 4, quad, 0)
            return carry
        lax.fori_loop(0, n_chunks // _AB, block, 0)

        plsc.subcore_barrier()

        rows = pl.ds(row_base, _RPT)
        pltpu.sync_copy(acc.at[rows], agg_out.at[c, rows])

    return k(xcat, src3, dst3)


def _sc_degree(dst6, half_chunks):
    f32 = jnp.float32
    mesh = plsc.VectorSubcoreMesh(core_axis_name="c", subcore_axis_name="s")

    @functools.partial(
        pl.kernel,
        out_type=jax.ShapeDtypeStruct((2, _NPAD, 128), f32),
        mesh=mesh,
        scratch_types=[
            pltpu.VMEM_SHARED((_NPAD, 128), f32),
            pltpu.VMEM((_IB, _CH), jnp.int32),
            pltpu.VMEM((_CH, 128), f32),
        ],
    )
    def k(dst_hbm, deg_out, acc, dst_v, ones_v):
        c = lax.axis_index("c")
        s = lax.axis_index("s")
        row_base = s * _RPT
        chunk_base = s * half_chunks

        def fill(val):
            def body(i, carry):
                for q in range(8):
                    ones_v[i, pl.ds(q * 16, 16)] = jnp.full((16,), val, f32)
                return carry
            lax.fori_loop(0, _CH, body, 0)

        fill(0.0)

        def zero_acc(j, carry):
            pltpu.sync_copy(ones_v, acc.at[pl.ds(row_base + j * _CH, _CH)])
            return carry
        lax.fori_loop(0, _RPT // _CH, zero_acc, 0)

        fill(1.0)
        plsc.subcore_barrier()

        def block(b, carry):
            pltpu.sync_copy(dst_hbm.at[c, pl.ds(chunk_base + b * _IB, _IB)], dst_v)
            for j in range(_IB):
                pltpu.sync_copy(ones_v, acc.at[dst_v.at[j]], add=True)
            return carry
        lax.fori_loop(0, half_chunks // _IB, block, 0)

        plsc.subcore_barrier()

        rows = pl.ds(row_base, _RPT)
        pltpu.sync_copy(acc.at[rows], deg_out.at[c, rows])

    return k(dst6)


def _prep_body(W1, Wg1, W2, Wg2, W3, b1, bg1, b2, bg2, b3,
               A1, B1, A2, B2, c):
    w3a = W3[0:128, :]
    w3b = W3[128:256, :]
    w3c = W3[256:384, :]
    w3d = W3[384:512, :]
    A1[...] = jnp.dot(W1[...], w3a, preferred_element_type=jnp.float32)
    B1[...] = jnp.dot(Wg1[...], w3b, preferred_element_type=jnp.float32)
    A2[...] = jnp.dot(W2[...], w3c, preferred_element_type=jnp.float32)
    B2[...] = jnp.dot(Wg2[...], w3d, preferred_element_type=jnp.float32)
    c[...] = (b3[...]
              + jnp.dot(b1[...], w3a, preferred_element_type=jnp.float32)
              + jnp.dot(bg1[...], w3b, preferred_element_type=jnp.float32)
              + jnp.dot(b2[...], w3c, preferred_element_type=jnp.float32)
              + jnp.dot(bg2[...], w3d, preferred_element_type=jnp.float32))


def _prep(W1, Wg1, W2, Wg2, W3, b1, bg1, b2, bg2, b3):
    f32 = jnp.float32
    return pl.pallas_call(
        _prep_body,
        out_shape=[
            jax.ShapeDtypeStruct((128, 256), f32),
            jax.ShapeDtypeStruct((128, 256), f32),
            jax.ShapeDtypeStruct((128, 256), f32),
            jax.ShapeDtypeStruct((128, 256), f32),
            jax.ShapeDtypeStruct((1, 256), f32),
        ],
    )(W1, Wg1, W2, Wg2, W3,
      b1.reshape(1, 128), bg1.reshape(1, 128),
      b2.reshape(1, 128), bg2.reshape(1, 128), b3.reshape(1, 256))


def _main_body(te, f2, agg, deg, A1, B1, A2, B2, c, out):
    r = 1.0 / jnp.maximum(deg[0] + deg[1], 1.0)
    acc = jnp.dot(te[...], A1[...], preferred_element_type=jnp.float32)
    acc += jnp.dot(f2[...], A2[...], preferred_element_type=jnp.float32)
    acc += jnp.dot(agg[0] * r, B1[...], preferred_element_type=jnp.float32)
    acc += jnp.dot(agg[1] * r, B2[...], preferred_element_type=jnp.float32)
    out[...] = acc + c[...]


def _fused_matmul(te, f2, agg, deg, A1, B1, A2, B2, c):
    n = te.shape[0]
    R = 2000
    grid = (n // R,)
    row_blk = pl.BlockSpec((R, 128), lambda i: (i, 0))
    pair_blk = pl.BlockSpec((2, R, 128), lambda i: (0, i, 0))
    w_blk = pl.BlockSpec((128, 256), lambda i: (0, 0))
    return pl.pallas_call(
        _main_body,
        grid=grid,
        in_specs=[row_blk, row_blk, pair_blk, pair_blk,
                  w_blk, w_blk, w_blk, w_blk,
                  pl.BlockSpec((1, 256), lambda i: (0, 0))],
        out_specs=pl.BlockSpec((R, 256), lambda i: (i, 0)),
        out_shape=jax.ShapeDtypeStruct((n, 256), jnp.float32),
    )(te, f2, agg, deg, A1, B1, A2, B2, c)


def kernel(text_emb, feature_2, edge_index, W_g1, b_g1, W_g2, b_g2,
           W1, b1, W2, b2, W3, b3):
    n = text_emb.shape[0]
    e = edge_index.shape[1]
    src = edge_index[0].astype(jnp.int32)
    dst = edge_index[1].astype(jnp.int32)

    n_chunks = -(-e // (16 * _CH))
    n_chunks = (n_chunks + _AB - 1) // _AB * _AB
    epad = 16 * n_chunks * _CH
    src_p = jnp.concatenate([src, jnp.zeros((epad - e,), jnp.int32)])
    dst_p = jnp.concatenate([dst, jnp.full((epad - e,), n, jnp.int32)])
    src3 = jnp.stack([src_p, src_p + n]).reshape(2, 16 * n_chunks, _CH)
    dst3 = dst_p.reshape(16 * n_chunks, _CH)
    dst6 = dst_p.reshape(2, 8 * n_chunks, _CH)
    xcat = jnp.concatenate([text_emb, feature_2], axis=0)

    agg = _sc_aggregate(xcat, src3, dst3, n_chunks)
    deg = _sc_degree(dst6, n_chunks // 2)

    A1, B1, A2, B2, c = _prep(W1, W_g1, W2, W_g2, W3, b1, b_g1, b2, b_g2, b3)
    return _fused_matmul(text_emb, feature_2, agg, deg, A1, B1, A2, B2, c)
